# baseline (speedup 1.0000x reference)
import functools

import jax
import jax.numpy as jnp
from jax import lax
from jax.experimental import pallas as pl
from jax.experimental.pallas import tpu as pltpu
from jax.experimental.pallas import tpu_sc as plsc

_N_NODES = 10000
_N_EDGES = 320000
_D_FEAT = 128

_NC = 2
_NS = 16
_NW = _NC * _NS
_EPT = _N_EDGES // _NW
_L = 16
_ALN = 128
_SZ = -(-_EPT // _ALN) * _ALN
_SPLIT = 5120
_SPLIT_AL = 5248


_QOFF = _N_NODES


def _tc_body(h_ref, w2_ref, b_ref, pq_ref):
    pq = lax.dot_general(
        w2_ref[...], h_ref[...],
        dimension_numbers=(((1,), (1,)), ((), ())),
        preferred_element_type=jnp.float32,
    )
    bias = 0.5 * b_ref[0]
    pq_ref[pl.ds(0, _N_NODES)] = pq[0, :] + bias
    pq_ref[pl.ds(_N_NODES, _N_NODES)] = pq[1, :] + bias


def _make_sc_kernel():
    mesh = plsc.VectorSubcoreMesh(core_axis_name="c", subcore_axis_name="s")

    @functools.partial(
        pl.kernel,
        mesh=mesh,
        out_type=jax.ShapeDtypeStruct((_N_EDGES,), jnp.float32),
        compiler_params=pltpu.CompilerParams(
            needs_layout_passes=False,
            disable_bounds_checks=True,
            disable_semaphore_checks=True,
            skip_device_barrier=True,
        ),
        scratch_types=[
            pltpu.VMEM((2 * _QOFF,), jnp.float32),
            pltpu.VMEM_SHARED((2 * _QOFF,), jnp.float32),
            pltpu.VMEM((2, _SZ), jnp.int32),
            pltpu.VMEM((_EPT,), jnp.float32),
            pltpu.SemaphoreType.DMA,
            pltpu.SemaphoreType.DMA,
        ],
    )
    def sc_k(pq_hbm, ei_hbm, out_hbm, pq_v, pq_sh, ei_v, o_v, sem, sem2):
        sid = lax.axis_index("s")
        wid = sid * _NC + lax.axis_index("c")
        base = wid * _EPT
        base_al = (base // _ALN) * _ALN
        off0 = base - base_al
        c2 = pltpu.async_copy(ei_hbm.at[:, pl.ds(base_al, _SPLIT_AL)],
                              ei_v.at[:, pl.ds(0, _SPLIT_AL)], sem)
        c3 = pltpu.async_copy(
            ei_hbm.at[:, pl.ds(base_al + _SPLIT_AL, _SZ - _SPLIT_AL)],
            ei_v.at[:, pl.ds(_SPLIT_AL, _SZ - _SPLIT_AL)], sem2)
        @pl.when(sid == 0)
        def _():
            pltpu.sync_copy(pq_hbm, pq_sh)
        plsc.subcore_barrier()
        pltpu.sync_copy(pq_sh, pq_v)
        c2.wait()

        def scores(off):
            sidx = ei_v[0, pl.ds(off0 + off, _L)]
            didx = ei_v[1, pl.ds(off0 + off, _L)]
            pv = plsc.load_gather(pq_v, [sidx])
            qv = plsc.load_gather(pq_v, [didx + _QOFF])
            x = pv + qv
            o_v[pl.ds(off, _L)] = 1.0 / (1.0 + jnp.exp(-x))

        plsc.parallel_loop(0, _SPLIT, step=_L, unroll=4)(scores)
        c3.wait()
        plsc.parallel_loop(_SPLIT, _EPT, step=_L, unroll=4)(scores)

        pltpu.sync_copy(o_v, out_hbm.at[pl.ds(base, _EPT)])

    return sc_k


_sc_kernel = _make_sc_kernel()


def kernel(h, edge_index, W, b):
    w2 = W.reshape(2, _D_FEAT)
    ei = edge_index.astype(jnp.int32)
    pq = pl.pallas_call(
        _tc_body,
        out_shape=jax.ShapeDtypeStruct((2 * _N_NODES,), jnp.float32),
        in_specs=[
            pl.BlockSpec(memory_space=pltpu.VMEM),
            pl.BlockSpec(memory_space=pltpu.VMEM),
            pl.BlockSpec(memory_space=pltpu.SMEM),
        ],
        out_specs=pl.BlockSpec(memory_space=pltpu.VMEM),
    )(h, w2, b)
    scores = _sc_kernel(pq, ei)
    out = scores.reshape(_N_EDGES, 1)
    return (out, out)


# speedup vs baseline: 1.1328x; 1.1328x over previous
import functools

import jax
import jax.numpy as jnp
from jax import lax
from jax.experimental import pallas as pl
from jax.experimental.pallas import tpu as pltpu
from jax.experimental.pallas import tpu_sc as plsc

_N_NODES = 10000
_N_EDGES = 320000
_D_FEAT = 128

_NC = 2
_NS = 16
_NW = _NC * _NS
_L = 16
_QOFF = _N_NODES

_EPT = 10240
_ROWS = _EPT // 128
_LAST = _N_EDGES - (_NW - 1) * _EPT
_NROW = _N_EDGES // 128


def _tc_body(h_ref, w2_ref, b_ref, pq_ref):
    pq = lax.dot_general(
        w2_ref[...], h_ref[...],
        dimension_numbers=(((1,), (1,)), ((), ())),
        preferred_element_type=jnp.float32,
    )
    bias = 0.5 * b_ref[0]
    pq_ref[pl.ds(0, _N_NODES)] = pq[0, :] + bias
    pq_ref[pl.ds(_N_NODES, _N_NODES)] = pq[1, :] + bias


def _make_sc_kernel():
    mesh = plsc.VectorSubcoreMesh(core_axis_name="c", subcore_axis_name="s")

    @functools.partial(
        pl.kernel,
        mesh=mesh,
        out_type=jax.ShapeDtypeStruct((_NROW, 1, 128), jnp.float32),
        compiler_params=pltpu.CompilerParams(needs_layout_passes=False),
        scratch_types=[
            pltpu.VMEM((2 * _QOFF,), jnp.float32),
            pltpu.VMEM_SHARED((2 * _QOFF,), jnp.float32),
            pltpu.VMEM((2, _EPT), jnp.int32),
            pltpu.VMEM((_ROWS, 1, 128), jnp.float32),
            pltpu.SemaphoreType.DMA,
        ],
    )
    def sc_k(pq_hbm, ei_hbm, out_hbm, pq_v, pq_sh, ei_v, o_v, sem):
        sid = lax.axis_index("s")
        wid = sid * _NC + lax.axis_index("c")
        last = wid == _NW - 1
        base = wid * _EPT
        base_ld = jnp.where(last, _N_EDGES - _EPT, base)
        off0 = base - base_ld
        c2 = pltpu.async_copy(ei_hbm.at[:, pl.ds(base_ld, _EPT)], ei_v, sem)
        @pl.when(sid == 0)
        def _():
            pltpu.sync_copy(pq_hbm, pq_sh)
        plsc.subcore_barrier()
        pltpu.sync_copy(pq_sh, pq_v)
        c2.wait()

        def scores(off):
            sidx = ei_v[0, pl.ds(off0 + off, _L)]
            didx = ei_v[1, pl.ds(off0 + off, _L)]
            pv = plsc.load_gather(pq_v, [sidx])
            qv = plsc.load_gather(pq_v, [didx + _QOFF])
            x = pv + qv
            r = lax.shift_right_logical(off, 7)
            c = lax.bitwise_and(off, 127)
            o_v[r, 0, pl.ds(c, _L)] = 1.0 / (1.0 + jnp.exp(-x))

        @pl.when(jnp.logical_not(last))
        def _():
            plsc.parallel_loop(0, _EPT, step=_L, unroll=4)(scores)
            pltpu.sync_copy(o_v, out_hbm.at[pl.ds(wid * _ROWS, _ROWS)])

        @pl.when(last)
        def _():
            plsc.parallel_loop(0, _LAST, step=_L, unroll=4)(scores)
            pltpu.sync_copy(
                o_v.at[pl.ds(0, _LAST // 128)],
                out_hbm.at[pl.ds(_NROW - _LAST // 128, _LAST // 128)])

    return sc_k


_sc_kernel = _make_sc_kernel()


def kernel(h, edge_index, W, b):
    w2 = W.reshape(2, _D_FEAT)
    ei = edge_index.astype(jnp.int32)
    pq = pl.pallas_call(
        _tc_body,
        out_shape=jax.ShapeDtypeStruct((2 * _N_NODES,), jnp.float32),
        in_specs=[
            pl.BlockSpec(memory_space=pltpu.VMEM),
            pl.BlockSpec(memory_space=pltpu.VMEM),
            pl.BlockSpec(memory_space=pltpu.SMEM),
        ],
        out_specs=pl.BlockSpec(memory_space=pltpu.VMEM),
    )(h, w2, b)
    scores = _sc_kernel(pq, ei)
    out = scores.reshape(_N_EDGES, 1)
    return (out, out)


# speedup vs baseline: 1.2575x; 1.1101x over previous
import functools

import jax
import jax.numpy as jnp
from jax import lax
from jax.experimental import pallas as pl
from jax.experimental.pallas import tpu as pltpu
from jax.experimental.pallas import tpu_sc as plsc

_N_NODES = 10000
_N_EDGES = 320000
_D_FEAT = 128

_NC = 2
_NS = 16
_NW = _NC * _NS
_L = 16
_QOFF = _N_NODES

_EPT = 10240
_ROWS = _EPT // 128
_LAST = _N_EDGES - (_NW - 1) * _EPT
_NROW = _N_EDGES // 128


def _tc_body(h_ref, w2_ref, b_ref, pq_ref):
    pq = lax.dot_general(
        w2_ref[...], h_ref[...],
        dimension_numbers=(((1,), (1,)), ((), ())),
        preferred_element_type=jnp.float32,
    )
    bias = 0.5 * b_ref[0]
    pq_ref[pl.ds(0, _N_NODES)] = pq[0, :] + bias
    pq_ref[pl.ds(_N_NODES, _N_NODES)] = pq[1, :] + bias


def _make_sc_kernel():
    mesh = plsc.VectorSubcoreMesh(core_axis_name="c", subcore_axis_name="s")

    @functools.partial(
        pl.kernel,
        mesh=mesh,
        out_type=[jax.ShapeDtypeStruct((_NROW, 1, 128), jnp.float32),
                  jax.ShapeDtypeStruct((_NROW, 1, 128), jnp.float32)],
        compiler_params=pltpu.CompilerParams(needs_layout_passes=False),
        scratch_types=[
            pltpu.VMEM((2 * _QOFF,), jnp.float32),
            pltpu.VMEM_SHARED((2 * _QOFF,), jnp.float32),
            pltpu.VMEM((2, _EPT), jnp.int32),
            pltpu.VMEM((_ROWS, 1, 128), jnp.float32),
            pltpu.SemaphoreType.DMA,
        ],
    )
    def sc_k(pq_hbm, ei_hbm, out_hbm, out2_hbm, pq_v, pq_sh, ei_v, o_v, sem):
        sid = lax.axis_index("s")
        wid = sid * _NC + lax.axis_index("c")
        last = wid == _NW - 1
        base = wid * _EPT
        base_ld = jnp.where(last, _N_EDGES - _EPT, base)
        off0 = base - base_ld
        c2 = pltpu.async_copy(ei_hbm.at[:, pl.ds(base_ld, _EPT)], ei_v, sem)
        @pl.when(sid == 0)
        def _():
            pltpu.sync_copy(pq_hbm, pq_sh)
        plsc.subcore_barrier()
        pltpu.sync_copy(pq_sh, pq_v)
        c2.wait()

        def scores(off):
            sidx = ei_v[0, pl.ds(off0 + off, _L)]
            didx = ei_v[1, pl.ds(off0 + off, _L)]
            pv = plsc.load_gather(pq_v, [sidx])
            qv = plsc.load_gather(pq_v, [didx + _QOFF])
            x = pv + qv
            r = lax.shift_right_logical(off, 7)
            c = lax.bitwise_and(off, 127)
            o_v[r, 0, pl.ds(c, _L)] = 1.0 / (1.0 + jnp.exp(-x))

        @pl.when(jnp.logical_not(last))
        def _():
            plsc.parallel_loop(0, _EPT, step=_L, unroll=4)(scores)
            w1 = pltpu.async_copy(
                o_v, out_hbm.at[pl.ds(wid * _ROWS, _ROWS)], sem)
            w2 = pltpu.async_copy(
                o_v, out2_hbm.at[pl.ds(wid * _ROWS, _ROWS)], sem)
            w1.wait()
            w2.wait()

        @pl.when(last)
        def _():
            plsc.parallel_loop(0, _LAST, step=_L, unroll=4)(scores)
            w1 = pltpu.async_copy(
                o_v.at[pl.ds(0, _LAST // 128)],
                out_hbm.at[pl.ds(_NROW - _LAST // 128, _LAST // 128)], sem)
            w2 = pltpu.async_copy(
                o_v.at[pl.ds(0, _LAST // 128)],
                out2_hbm.at[pl.ds(_NROW - _LAST // 128, _LAST // 128)], sem)
            w1.wait()
            w2.wait()

    return sc_k


_sc_kernel = _make_sc_kernel()


def kernel(h, edge_index, W, b):
    w2 = W.reshape(2, _D_FEAT)
    ei = edge_index.astype(jnp.int32)
    pq = pl.pallas_call(
        _tc_body,
        out_shape=jax.ShapeDtypeStruct((2 * _N_NODES,), jnp.float32),
        in_specs=[
            pl.BlockSpec(memory_space=pltpu.VMEM),
            pl.BlockSpec(memory_space=pltpu.VMEM),
            pl.BlockSpec(memory_space=pltpu.SMEM),
        ],
        out_specs=pl.BlockSpec(memory_space=pltpu.VMEM),
    )(h, w2, b)
    s1, s2 = _sc_kernel(pq, ei)
    return (s1.reshape(_N_EDGES, 1), s2.reshape(_N_EDGES, 1))
